# Initial kernel scaffold; baseline (speedup 1.0000x reference)
import jax
import jax.numpy as jnp
from jax.experimental import pallas as pl


def kernel(query_len, key_len, bias_table):
    raise NotImplementedError("write your pallas kernel here")


# speedup vs baseline: 42.4047x; 42.4047x over previous
import functools
import math

import jax
import jax.numpy as jnp
from jax import lax
from jax.experimental import pallas as pl
from jax.experimental.pallas import tpu as pltpu
from jax.experimental.pallas import tpu_sc as plsc

NUM_BUCKETS = 32
MAX_DISTANCE = 128
NUM_HEADS = 16
QUERY_LEN = 2048
KEY_LEN = 2048

P = 16
PHASE = 15
W = 4096
HALF_Q = QUERY_LEN // 2
BLOCKS = HALF_Q // P


def _build_shifted_table_kernel(tbl_ref, s_ref):
    half = NUM_BUCKETS // 2
    max_exact = half // 2
    scale = (half - max_exact) / math.log(MAX_DISTANCE / max_exact)
    i_iota = lax.broadcasted_iota(jnp.int32, (NUM_BUCKETS, W), 1)
    b_iota = lax.broadcasted_iota(jnp.int32, (NUM_BUCKETS, W), 0)
    tbl = tbl_ref[...]
    for r in range(P):
        n = (QUERY_LEN - 1 - PHASE + r) - i_iota
        ret = jnp.where(n < 0, half, 0)
        na = jnp.abs(n)
        is_small = na < max_exact
        safe = jnp.maximum(na, 1)
        log_val = (jnp.log(safe.astype(jnp.float32) / max_exact) * scale)
        log_val = log_val.astype(jnp.int32)
        bucket = jnp.where(is_small, na, max_exact + log_val)
        bucket = jnp.clip(bucket, 0, half - 1) + ret
        one_hot = (bucket == b_iota).astype(jnp.float32)
        row = lax.dot_general(tbl, one_hot, (((0,), (0,)), ((), ())),
                              preferred_element_type=jnp.float32)
        s_ref[:, r, :] = row


def _build_shifted_table(bias_table):
    return pl.pallas_call(
        _build_shifted_table_kernel,
        out_shape=jax.ShapeDtypeStruct((NUM_HEADS, P, W), jnp.float32),
    )(bias_table)


def _expand(s):
    mesh = plsc.VectorSubcoreMesh(core_axis_name="c", subcore_axis_name="s")

    @functools.partial(
        pl.kernel,
        out_type=jax.ShapeDtypeStruct((NUM_HEADS, QUERY_LEN, KEY_LEN),
                                      jnp.float32),
        mesh=mesh,
        scratch_types=[pltpu.VMEM((P, W), jnp.float32)],
        compiler_params=pltpu.CompilerParams(use_tc_tiling_on_sc=False),
    )
    def expand_kernel(s_hbm, out_hbm, buf):
        wid = lax.axis_index("c") * 16 + lax.axis_index("s")
        h = wid // 2
        q_base = (wid % 2) * HALF_Q
        pltpu.sync_copy(s_hbm.at[h], buf)

        @pl.loop(0, BLOCKS)
        def _(b):
            q0 = q_base + b * P
            i0 = (QUERY_LEN - 1 - PHASE) - q0
            pltpu.sync_copy(buf.at[:, pl.ds(i0, KEY_LEN)],
                            out_hbm.at[h, pl.ds(q0, P), :])

    return expand_kernel(s)


def kernel(query_len, key_len, bias_table):
    del query_len, key_len
    s = _build_shifted_table(bias_table)
    out = _expand(s)
    return out[None]


# speedup vs baseline: 42.4755x; 1.0017x over previous
import functools
import math

import jax
import jax.numpy as jnp
from jax import lax
from jax.experimental import pallas as pl
from jax.experimental.pallas import tpu as pltpu
from jax.experimental.pallas import tpu_sc as plsc

NUM_BUCKETS = 32
MAX_DISTANCE = 128
NUM_HEADS = 16
QUERY_LEN = 2048
KEY_LEN = 2048

P = 16
PHASE = 15
W = 4096
HALF_Q = QUERY_LEN // 2
BLOCKS = HALF_Q // P


def _build_shifted_table_kernel(tbl_ref, s_ref):
    half = NUM_BUCKETS // 2
    max_exact = half // 2
    scale = (half - max_exact) / math.log(MAX_DISTANCE / max_exact)
    i_iota = lax.broadcasted_iota(jnp.int32, (NUM_BUCKETS, W), 1)
    b_iota = lax.broadcasted_iota(jnp.int32, (NUM_BUCKETS, W), 0)
    tbl = tbl_ref[...]
    for r in range(P):
        n = (QUERY_LEN - 1 - PHASE + r) - i_iota
        ret = jnp.where(n < 0, half, 0)
        na = jnp.abs(n)
        is_small = na < max_exact
        safe = jnp.maximum(na, 1)
        log_val = (jnp.log(safe.astype(jnp.float32) / max_exact) * scale)
        log_val = log_val.astype(jnp.int32)
        bucket = jnp.where(is_small, na, max_exact + log_val)
        bucket = jnp.clip(bucket, 0, half - 1) + ret
        one_hot = (bucket == b_iota).astype(jnp.float32)
        row = lax.dot_general(tbl, one_hot, (((0,), (0,)), ((), ())),
                              preferred_element_type=jnp.float32)
        s_ref[:, r, :] = row


def _build_shifted_table(bias_table):
    return pl.pallas_call(
        _build_shifted_table_kernel,
        out_shape=jax.ShapeDtypeStruct((NUM_HEADS, P, W), jnp.float32),
    )(bias_table)


def _expand(s):
    mesh = plsc.VectorSubcoreMesh(core_axis_name="c", subcore_axis_name="s")

    @functools.partial(
        pl.kernel,
        out_type=jax.ShapeDtypeStruct((NUM_HEADS, QUERY_LEN, KEY_LEN),
                                      jnp.float32),
        mesh=mesh,
        scratch_types=[pltpu.VMEM((P, W), jnp.float32),
                       pltpu.SemaphoreType.DMA],
        compiler_params=pltpu.CompilerParams(use_tc_tiling_on_sc=False),
    )
    def expand_kernel(s_hbm, out_hbm, buf, sem):
        wid = lax.axis_index("c") * 16 + lax.axis_index("s")
        h = wid // 2
        q_base = (wid % 2) * HALF_Q
        pltpu.sync_copy(s_hbm.at[h], buf)

        DEPTH = 4

        def block_refs(q0):
            i0 = (QUERY_LEN - 1 - PHASE) - q0
            return (buf.at[:, pl.ds(i0, KEY_LEN)],
                    out_hbm.at[h, pl.ds(q0, P), :])

        @pl.loop(0, BLOCKS)
        def _(b):
            src, dst = block_refs(q_base + b * P)
            pltpu.async_copy(src, dst, sem)

            @pl.when(b >= DEPTH)
            def _():
                wsrc, wdst = block_refs(q_base)
                pltpu.make_async_copy(wsrc, wdst, sem).wait()

        for _ in range(DEPTH):
            wsrc, wdst = block_refs(q_base)
            pltpu.make_async_copy(wsrc, wdst, sem).wait()

    return expand_kernel(s)


def kernel(query_len, key_len, bias_table):
    del query_len, key_len
    s = _build_shifted_table(bias_table)
    out = _expand(s)
    return out[None]


# speedup vs baseline: 42.5638x; 1.0021x over previous
import functools
import math

import jax
import jax.numpy as jnp
from jax import lax
from jax.experimental import pallas as pl
from jax.experimental.pallas import tpu as pltpu
from jax.experimental.pallas import tpu_sc as plsc

NUM_BUCKETS = 32
MAX_DISTANCE = 128
NUM_HEADS = 16
QUERY_LEN = 2048
KEY_LEN = 2048

P = 16
PHASE = 15
W = 4096
HALF_Q = QUERY_LEN // 2
BLOCKS = HALF_Q // P


def _build_shifted_table_kernel(tbl_ref, s_ref):
    half = NUM_BUCKETS // 2
    max_exact = half // 2
    scale = (half - max_exact) / math.log(MAX_DISTANCE / max_exact)
    i_iota = lax.broadcasted_iota(jnp.int32, (NUM_BUCKETS, W), 1)
    b_iota = lax.broadcasted_iota(jnp.int32, (NUM_BUCKETS, W), 0)
    tbl = tbl_ref[...]
    for r in range(P):
        n = (QUERY_LEN - 1 - PHASE + r) - i_iota
        ret = jnp.where(n < 0, half, 0)
        na = jnp.abs(n)
        is_small = na < max_exact
        safe = jnp.maximum(na, 1)
        log_val = (jnp.log(safe.astype(jnp.float32) / max_exact) * scale)
        log_val = log_val.astype(jnp.int32)
        bucket = jnp.where(is_small, na, max_exact + log_val)
        bucket = jnp.clip(bucket, 0, half - 1) + ret
        one_hot = (bucket == b_iota).astype(jnp.float32)
        row = lax.dot_general(tbl, one_hot, (((0,), (0,)), ((), ())),
                              preferred_element_type=jnp.float32)
        s_ref[:, r, :] = row


def _build_shifted_table(bias_table):
    return pl.pallas_call(
        _build_shifted_table_kernel,
        out_shape=jax.ShapeDtypeStruct((NUM_HEADS, P, W), jnp.float32),
    )(bias_table)


def _expand(s):
    mesh = plsc.VectorSubcoreMesh(core_axis_name="c", subcore_axis_name="s")

    @functools.partial(
        pl.kernel,
        out_type=jax.ShapeDtypeStruct((1, NUM_HEADS, QUERY_LEN, KEY_LEN),
                                      jnp.float32),
        mesh=mesh,
        scratch_types=[pltpu.VMEM((P, W), jnp.float32),
                       pltpu.SemaphoreType.DMA],
        compiler_params=pltpu.CompilerParams(use_tc_tiling_on_sc=False),
    )
    def expand_kernel(s_hbm, out_hbm, buf, sem):
        wid = lax.axis_index("c") * 16 + lax.axis_index("s")
        h = wid // 2
        q_base = (wid % 2) * HALF_Q
        pltpu.sync_copy(s_hbm.at[h], buf)

        DEPTH = 4

        def block_refs(q0):
            i0 = (QUERY_LEN - 1 - PHASE) - q0
            return (buf.at[:, pl.ds(i0, KEY_LEN)],
                    out_hbm.at[0, h, pl.ds(q0, P), :])

        @pl.loop(0, BLOCKS)
        def _(b):
            src, dst = block_refs(q_base + b * P)
            pltpu.async_copy(src, dst, sem)

            @pl.when(b >= DEPTH)
            def _():
                wsrc, wdst = block_refs(q_base)
                pltpu.make_async_copy(wsrc, wdst, sem).wait()

        for _ in range(DEPTH):
            wsrc, wdst = block_refs(q_base)
            pltpu.make_async_copy(wsrc, wdst, sem).wait()

    return expand_kernel(s)


def kernel(query_len, key_len, bias_table):
    del query_len, key_len
    s = _build_shifted_table(bias_table)
    return _expand(s)


# speedup vs baseline: 113.3440x; 2.6629x over previous
import functools
import math

import jax
import jax.numpy as jnp
from jax import lax
from jax.experimental import pallas as pl
from jax.experimental.pallas import tpu as pltpu
from jax.experimental.pallas import tpu_sc as plsc

NUM_BUCKETS = 32
MAX_DISTANCE = 128
NUM_HEADS = 16
QUERY_LEN = 2048
KEY_LEN = 2048

NPHASE = 128
W = 4096
TSW = 3968
UB = 8
P = 16
CLASSES = 8
UNITS_PER_TILE = 4
BLOCKS_PER_UNIT = QUERY_LEN // NPHASE


def _build_table_kernel(tbl_ref, s_ref):
    half = NUM_BUCKETS // 2
    max_exact = half // 2
    scale = (half - max_exact) / math.log(MAX_DISTANCE / max_exact)
    ub = pl.program_id(0)
    i_iota = lax.broadcasted_iota(jnp.int32, (NUM_BUCKETS, W), 1)
    b_iota = lax.broadcasted_iota(jnp.int32, (NUM_BUCKETS, W), 0)
    tbl = tbl_ref[...]
    for r in range(UB):
        n = (1920 + r - i_iota) + ub * UB
        ret = jnp.where(n < 0, half, 0)
        na = jnp.abs(n)
        is_small = na < max_exact
        safe = jnp.maximum(na, 1)
        log_val = (jnp.log(safe.astype(jnp.float32) / max_exact) * scale)
        log_val = log_val.astype(jnp.int32)
        bucket = jnp.where(is_small, na, max_exact + log_val)
        bucket = jnp.clip(bucket, 0, half - 1) + ret
        one_hot = (bucket == b_iota).astype(jnp.float32)
        row = lax.dot_general(tbl, one_hot, (((0,), (0,)), ((), ())),
                              preferred_element_type=jnp.float32)
        s_ref[:, r, :] = row


def _build_table(bias_table):
    return pl.pallas_call(
        _build_table_kernel,
        grid=(NPHASE // UB,),
        in_specs=[pl.BlockSpec((NUM_BUCKETS, NUM_HEADS), lambda ub: (0, 0))],
        out_specs=pl.BlockSpec((NUM_HEADS, UB, W), lambda ub: (0, ub, 0)),
        out_shape=jax.ShapeDtypeStruct((NUM_HEADS, NPHASE, W), jnp.float32),
    )(bias_table)


def _expand(sr):
    mesh = plsc.VectorSubcoreMesh(core_axis_name="c", subcore_axis_name="s")

    @functools.partial(
        pl.kernel,
        out_type=jax.ShapeDtypeStruct((1, NUM_HEADS, QUERY_LEN, KEY_LEN),
                                      jnp.float32),
        mesh=mesh,
        scratch_types=[pltpu.VMEM((P, TSW), jnp.float32),
                       pltpu.VMEM((P, TSW), jnp.float32),
                       pltpu.SemaphoreType.DMA,
                       pltpu.SemaphoreType.DMA],
    )
    def expand_kernel(sr_hbm, out_hbm, buf0, buf1, sem0, sem1):
        wid = lax.axis_index("c") * 16 + lax.axis_index("s")
        bufs = (buf0, buf1)
        sems = (sem0, sem1)

        def drain_unit(b, sm):
            for _ in range(BLOCKS_PER_UNIT):
                pltpu.make_async_copy(b.at[:, pl.ds(0, KEY_LEN)],
                                      out_hbm.at[0, 0, pl.ds(0, P), :],
                                      sm).wait()

        for t in range(UNITS_PER_TILE):
            unit = wid * UNITS_PER_TILE + t
            h = unit // CLASSES
            c = (unit % CLASSES) * P
            b = bufs[t % 2]
            sm = sems[t % 2]
            if t >= 2:
                drain_unit(b, sm)
            pltpu.sync_copy(sr_hbm.at[h, pl.ds(c, P), pl.ds(0, TSW)], b)
            for m in range(BLOCKS_PER_UNIT):
                i0 = 1920 - 128 * m
                q0 = c + 128 * m
                pltpu.async_copy(b.at[:, pl.ds(i0, KEY_LEN)],
                                 out_hbm.at[0, h, pl.ds(q0, P), :], sm)

        for t in range(2):
            drain_unit(bufs[t], sems[t])

    return expand_kernel(sr)


def kernel(query_len, key_len, bias_table):
    del query_len, key_len
    sr = _build_table(bias_table)
    return _expand(sr)


# speedup vs baseline: 115.6289x; 1.0202x over previous
import functools
import math

import jax
import jax.numpy as jnp
from jax import lax
from jax.experimental import pallas as pl
from jax.experimental.pallas import tpu as pltpu
from jax.experimental.pallas import tpu_sc as plsc

NUM_BUCKETS = 32
MAX_DISTANCE = 128
NUM_HEADS = 16
QUERY_LEN = 2048
KEY_LEN = 2048

NPHASE = 128
W = 4096
TSW = 3968
UB = 8
P = 16
CLASSES = 8
UNITS_PER_TILE = 4
BLOCKS_PER_UNIT = QUERY_LEN // NPHASE


VW = 4224


def _build_table_kernel(tbl_ref, s_ref):
    half = NUM_BUCKETS // 2
    max_exact = half // 2
    scale = (half - max_exact) / math.log(MAX_DISTANCE / max_exact)
    j_iota = lax.broadcasted_iota(jnp.int32, (NUM_BUCKETS, VW), 1)
    b_iota = lax.broadcasted_iota(jnp.int32, (NUM_BUCKETS, VW), 0)
    n = 2047 - j_iota
    ret = jnp.where(n < 0, half, 0)
    na = jnp.abs(n)
    is_small = na < max_exact
    safe = jnp.maximum(na, 1)
    log_val = (jnp.log(safe.astype(jnp.float32) / max_exact) * scale)
    log_val = log_val.astype(jnp.int32)
    bucket = jnp.where(is_small, na, max_exact + log_val)
    bucket = jnp.clip(bucket, 0, half - 1) + ret
    one_hot = (bucket == b_iota).astype(jnp.float32)
    v = lax.dot_general(tbl_ref[...], one_hot, (((0,), (0,)), ((), ())),
                        preferred_element_type=jnp.float32)
    for u0 in range(0, NPHASE, UB):
        rows = [lax.slice(v, (0, 127 - (u0 + r)), (NUM_HEADS, 127 - (u0 + r) + W))
                for r in range(UB)]
        s_ref[:, u0:u0 + UB, :] = jnp.stack(rows, axis=1)


def _build_table(bias_table):
    return pl.pallas_call(
        _build_table_kernel,
        out_shape=jax.ShapeDtypeStruct((NUM_HEADS, NPHASE, W), jnp.float32),
        compiler_params=pltpu.CompilerParams(vmem_limit_bytes=100 << 20),
    )(bias_table)


def _expand(sr):
    mesh = plsc.VectorSubcoreMesh(core_axis_name="c", subcore_axis_name="s")

    @functools.partial(
        pl.kernel,
        out_type=jax.ShapeDtypeStruct((1, NUM_HEADS, QUERY_LEN, KEY_LEN),
                                      jnp.float32),
        mesh=mesh,
        scratch_types=[pltpu.VMEM((P, TSW), jnp.float32),
                       pltpu.VMEM((P, TSW), jnp.float32),
                       pltpu.SemaphoreType.DMA,
                       pltpu.SemaphoreType.DMA],
    )
    def expand_kernel(sr_hbm, out_hbm, buf0, buf1, sem0, sem1):
        wid = lax.axis_index("c") * 16 + lax.axis_index("s")
        bufs = (buf0, buf1)
        sems = (sem0, sem1)

        def drain_unit(b, sm):
            for _ in range(BLOCKS_PER_UNIT):
                pltpu.make_async_copy(b.at[:, pl.ds(0, KEY_LEN)],
                                      out_hbm.at[0, 0, pl.ds(0, P), :],
                                      sm).wait()

        for t in range(UNITS_PER_TILE):
            unit = wid * UNITS_PER_TILE + t
            h = unit // CLASSES
            c = (unit % CLASSES) * P
            b = bufs[t % 2]
            sm = sems[t % 2]
            if t >= 2:
                drain_unit(b, sm)
            pltpu.sync_copy(sr_hbm.at[h, pl.ds(c, P), pl.ds(0, TSW)], b)
            for m in range(BLOCKS_PER_UNIT):
                i0 = 1920 - 128 * m
                q0 = c + 128 * m
                pltpu.async_copy(b.at[:, pl.ds(i0, KEY_LEN)],
                                 out_hbm.at[0, h, pl.ds(q0, P), :], sm)

        for t in range(2):
            drain_unit(bufs[t], sems[t])

    return expand_kernel(sr)


def kernel(query_len, key_len, bias_table):
    del query_len, key_len
    sr = _build_table(bias_table)
    return _expand(sr)


# speedup vs baseline: 119.0089x; 1.0292x over previous
import functools
import math

import jax
import jax.numpy as jnp
from jax import lax
from jax.experimental import pallas as pl
from jax.experimental.pallas import tpu as pltpu
from jax.experimental.pallas import tpu_sc as plsc

NUM_BUCKETS = 32
MAX_DISTANCE = 128
NUM_HEADS = 16
QUERY_LEN = 2048
KEY_LEN = 2048

NPHASE = 128
W = 4096
TSW = 3968
UB = 8
P = 16
CLASSES = 8
UNITS_PER_TILE = 4
BLOCKS_PER_UNIT = QUERY_LEN // NPHASE


def _build_table_kernel(tbl_ref, s_ref):
    half = NUM_BUCKETS // 2
    max_exact = half // 2
    scale = (half - max_exact) / math.log(MAX_DISTANCE / max_exact)
    ub = pl.program_id(0)
    r_iota = lax.broadcasted_iota(jnp.int32, (UB, W), 0)
    i_iota = lax.broadcasted_iota(jnp.int32, (UB, W), 1)
    b_iota = lax.broadcasted_iota(jnp.int32, (NUM_BUCKETS, W), 0)
    n = (1920 + r_iota - i_iota) + ub * UB
    ret = jnp.where(n < 0, half, 0)
    na = jnp.abs(n)
    is_small = na < max_exact
    safe = jnp.maximum(na, 1)
    log_val = (jnp.log(safe.astype(jnp.float32) / max_exact) * scale)
    log_val = log_val.astype(jnp.int32)
    bucket = jnp.where(is_small, na, max_exact + log_val)
    bucket = jnp.clip(bucket, 0, half - 1) + ret
    tbl = tbl_ref[...]
    for r in range(UB):
        row_b = jnp.broadcast_to(lax.slice(bucket, (r, 0), (r + 1, W)),
                                 (NUM_BUCKETS, W))
        one_hot = (row_b == b_iota).astype(jnp.float32)
        row = lax.dot_general(tbl, one_hot, (((0,), (0,)), ((), ())),
                              preferred_element_type=jnp.float32)
        s_ref[:, r, :] = row


def _build_table(bias_table):
    return pl.pallas_call(
        _build_table_kernel,
        grid=(NPHASE // UB,),
        in_specs=[pl.BlockSpec((NUM_BUCKETS, NUM_HEADS), lambda ub: (0, 0))],
        out_specs=pl.BlockSpec((NUM_HEADS, UB, W), lambda ub: (0, ub, 0)),
        out_shape=jax.ShapeDtypeStruct((NUM_HEADS, NPHASE, W), jnp.float32),
    )(bias_table)


def _expand(sr):
    mesh = plsc.VectorSubcoreMesh(core_axis_name="c", subcore_axis_name="s")

    @functools.partial(
        pl.kernel,
        out_type=jax.ShapeDtypeStruct((1, NUM_HEADS, QUERY_LEN, KEY_LEN),
                                      jnp.float32),
        mesh=mesh,
        scratch_types=[pltpu.VMEM((P, TSW), jnp.float32),
                       pltpu.VMEM((P, TSW), jnp.float32),
                       pltpu.SemaphoreType.DMA,
                       pltpu.SemaphoreType.DMA],
    )
    def expand_kernel(sr_hbm, out_hbm, buf0, buf1, sem0, sem1):
        wid = lax.axis_index("c") * 16 + lax.axis_index("s")
        bufs = (buf0, buf1)
        sems = (sem0, sem1)

        def drain_unit(b, sm):
            for _ in range(BLOCKS_PER_UNIT):
                pltpu.make_async_copy(b.at[:, pl.ds(0, KEY_LEN)],
                                      out_hbm.at[0, 0, pl.ds(0, P), :],
                                      sm).wait()

        for t in range(UNITS_PER_TILE):
            unit = wid * UNITS_PER_TILE + t
            h = unit // CLASSES
            c = (unit % CLASSES) * P
            b = bufs[t % 2]
            sm = sems[t % 2]
            if t >= 2:
                drain_unit(b, sm)
            pltpu.sync_copy(sr_hbm.at[h, pl.ds(c, P), pl.ds(0, TSW)], b)
            for m in range(BLOCKS_PER_UNIT):
                i0 = 1920 - 128 * m
                q0 = c + 128 * m
                pltpu.async_copy(b.at[:, pl.ds(i0, KEY_LEN)],
                                 out_hbm.at[0, h, pl.ds(q0, P), :], sm)

        for t in range(2):
            drain_unit(bufs[t], sems[t])

    return expand_kernel(sr)


def kernel(query_len, key_len, bias_table):
    del query_len, key_len
    sr = _build_table(bias_table)
    return _expand(sr)
